# baseline (speedup 1.0000x reference)
import jax
import jax.numpy as jnp
from jax import lax
from jax.experimental import pallas as pl
from jax.experimental.pallas import tpu as pltpu
from jax.experimental.pallas import tpu_sc as plsc

D_HALF = 64
D = 2 * D_HALF
BATCH = 16384

_info = plsc.get_sparse_core_info()
_NC, _NS = _info.num_cores, _info.num_subcores
_NW = _NC * _NS
_B_PER_W = BATCH // _NW
_CHUNK = 128
_N_CHUNKS = _B_PER_W // _CHUNK


def _pe_body(x_hbm, y_hbm, peh_hbm, pew_hbm, out_hbm, idx_x, idx_y, rows,
             semi_x, semi_y, *sems):
    sx = sems[:_N_CHUNKS]
    sy = sems[_N_CHUNKS:2 * _N_CHUNKS]
    sw = sems[2 * _N_CHUNKS]
    wid = lax.axis_index("s") * _NC + lax.axis_index("c")
    base = wid * _B_PER_W
    ix = pltpu.async_copy(x_hbm.at[pl.ds(base, _B_PER_W)], idx_x, semi_x)
    iy = pltpu.async_copy(y_hbm.at[pl.ds(base, _B_PER_W)], idx_y, semi_y)
    ix.wait()
    xs = []
    for c in range(_N_CHUNKS):
        off = c * _CHUNK
        xs.append(pltpu.async_copy(
            pew_hbm.at[idx_x.at[pl.ds(off, _CHUNK)]],
            rows.at[pl.ds(off, _CHUNK)], sx[c]))
    iy.wait()
    ys = []
    for c in range(_N_CHUNKS):
        off = c * _CHUNK
        xs[c].wait()
        ys.append(pltpu.async_copy(
            peh_hbm.at[idx_y.at[pl.ds(off, _CHUNK)]],
            rows.at[pl.ds(off, _CHUNK)], sy[c], add=True))
    ws = []
    for c in range(_N_CHUNKS):
        off = c * _CHUNK
        ys[c].wait()
        ws.append(pltpu.async_copy(
            rows.at[pl.ds(off, _CHUNK)],
            out_hbm.at[pl.ds(base + off, _CHUNK)], sw))
    for w in ws:
        w.wait()


@jax.jit
def _pe_kernel(x, y, peh_wide, pew_wide):
    mesh = plsc.VectorSubcoreMesh(core_axis_name="c", subcore_axis_name="s")
    return pl.kernel(
        _pe_body,
        out_type=jax.ShapeDtypeStruct((BATCH, D), jnp.float32),
        mesh=mesh,
        scratch_types=(
            [
                pltpu.VMEM((_B_PER_W,), jnp.int32),
                pltpu.VMEM((_B_PER_W,), jnp.int32),
                pltpu.VMEM((_B_PER_W, D), jnp.float32),
            ]
            + [pltpu.SemaphoreType.DMA] * (2 + 2 * _N_CHUNKS + 1)
        ),
    )(x, y, peh_wide, pew_wide)


def kernel(x, y, pe_h, pe_w):
    x = x.astype(jnp.int32)
    y = y.astype(jnp.int32)
    zeros = jnp.zeros_like(pe_w)
    pew_wide = jnp.concatenate([pe_w, zeros], axis=1)
    peh_wide = jnp.concatenate([zeros, pe_h], axis=1)
    return _pe_kernel(x, y, peh_wide, pew_wide)


# speedup vs baseline: 1.2406x; 1.2406x over previous
import jax
import jax.numpy as jnp
from jax import lax
from jax.experimental import pallas as pl
from jax.experimental.pallas import tpu as pltpu
from jax.experimental.pallas import tpu_sc as plsc

D_HALF = 64
D = 2 * D_HALF
BATCH = 16384
TABLE_ROWS = 1000

_info = plsc.get_sparse_core_info()
_NC, _NS = _info.num_cores, _info.num_subcores
_NW = _NC * _NS
_B_PER_W = BATCH // _NW
_CHUNK = 128
_N_CHUNKS = _B_PER_W // _CHUNK
_STAGE_ROWS = TABLE_ROWS // 5


def _pe_body(x_hbm, y_hbm, peh_hbm, pew_hbm, out_hbm,
             idx_x, idx_y, rows, pew_sp, peh_sp, semi_x, semi_y, *sems):
    sx = sems[:_N_CHUNKS]
    sy = sems[_N_CHUNKS:2 * _N_CHUNKS]
    sw = sems[2 * _N_CHUNKS]
    sid = lax.axis_index("s")
    wid = sid * _NC + lax.axis_index("c")
    base = wid * _B_PER_W
    ix = pltpu.async_copy(x_hbm.at[pl.ds(base, _B_PER_W)], idx_x, semi_x)
    iy = pltpu.async_copy(y_hbm.at[pl.ds(base, _B_PER_W)], idx_y, semi_y)
    row0 = (sid % 8) * _STAGE_ROWS

    @pl.when(sid < 5)
    def _():
        pltpu.sync_copy(pew_hbm.at[pl.ds(row0, _STAGE_ROWS)],
                        pew_sp.at[pl.ds(row0, _STAGE_ROWS)])

    @pl.when(jnp.logical_and(sid >= 8, sid < 13))
    def _():
        pltpu.sync_copy(peh_hbm.at[pl.ds(row0, _STAGE_ROWS)],
                        peh_sp.at[pl.ds(row0, _STAGE_ROWS)])

    plsc.subcore_barrier()
    ix.wait()
    xs = []
    for c in range(_N_CHUNKS):
        off = c * _CHUNK
        xs.append(pltpu.async_copy(
            pew_sp.at[idx_x.at[pl.ds(off, _CHUNK)]],
            rows.at[pl.ds(off, _CHUNK)], sx[c]))
    iy.wait()
    ys = []
    for c in range(_N_CHUNKS):
        off = c * _CHUNK
        xs[c].wait()
        ys.append(pltpu.async_copy(
            peh_sp.at[idx_y.at[pl.ds(off, _CHUNK)]],
            rows.at[pl.ds(off, _CHUNK)], sy[c], add=True))
    ws = []
    for c in range(_N_CHUNKS):
        off = c * _CHUNK
        ys[c].wait()
        ws.append(pltpu.async_copy(
            rows.at[pl.ds(off, _CHUNK)],
            out_hbm.at[pl.ds(base + off, _CHUNK)], sw))
    for w in ws:
        w.wait()


@jax.jit
def _pe_kernel(x, y, peh_wide, pew_wide):
    mesh = plsc.VectorSubcoreMesh(core_axis_name="c", subcore_axis_name="s")
    return pl.kernel(
        _pe_body,
        out_type=jax.ShapeDtypeStruct((BATCH, D), jnp.float32),
        mesh=mesh,
        scratch_types=(
            [
                pltpu.VMEM((_B_PER_W,), jnp.int32),
                pltpu.VMEM((_B_PER_W,), jnp.int32),
                pltpu.VMEM((_B_PER_W, D), jnp.float32),
                pltpu.VMEM_SHARED((TABLE_ROWS, D), jnp.float32),
                pltpu.VMEM_SHARED((TABLE_ROWS, D), jnp.float32),
            ]
            + [pltpu.SemaphoreType.DMA] * (2 + 2 * _N_CHUNKS + 1)
        ),
    )(x, y, peh_wide, pew_wide)


def kernel(x, y, pe_h, pe_w):
    x = x.astype(jnp.int32)
    y = y.astype(jnp.int32)
    zeros = jnp.zeros_like(pe_w)
    pew_wide = jnp.concatenate([pe_w, zeros], axis=1)
    peh_wide = jnp.concatenate([zeros, pe_h], axis=1)
    return _pe_kernel(x, y, peh_wide, pew_wide)


# speedup vs baseline: 1.2446x; 1.0032x over previous
import jax
import jax.numpy as jnp
from jax import lax
from jax.experimental import pallas as pl
from jax.experimental.pallas import tpu as pltpu
from jax.experimental.pallas import tpu_sc as plsc

D_HALF = 64
D = 2 * D_HALF
BATCH = 16384
TABLE_ROWS = 1000

_info = plsc.get_sparse_core_info()
_NC, _NS = _info.num_cores, _info.num_subcores
_NW = _NC * _NS
_B_PER_W = BATCH // _NW
_CHUNK = 64
_N_CHUNKS = _B_PER_W // _CHUNK
_STAGE_ROWS = TABLE_ROWS // 5


def _pe_body(x_hbm, y_hbm, peh_hbm, pew_hbm, out_hbm,
             idx_x, idx_y, rows, pew_sp, peh_sp, semi_x, semi_y, *sems):
    sx = sems[:_N_CHUNKS]
    sy = sems[_N_CHUNKS:2 * _N_CHUNKS]
    sw = sems[2 * _N_CHUNKS]
    sid = lax.axis_index("s")
    wid = sid * _NC + lax.axis_index("c")
    base = wid * _B_PER_W
    ix = pltpu.async_copy(x_hbm.at[pl.ds(base, _B_PER_W)], idx_x, semi_x)
    iy = pltpu.async_copy(y_hbm.at[pl.ds(base, _B_PER_W)], idx_y, semi_y)
    row0 = (sid % 8) * _STAGE_ROWS

    @pl.when(sid < 5)
    def _():
        pltpu.sync_copy(pew_hbm.at[pl.ds(row0, _STAGE_ROWS)],
                        pew_sp.at[pl.ds(row0, _STAGE_ROWS)])

    @pl.when(jnp.logical_and(sid >= 8, sid < 13))
    def _():
        pltpu.sync_copy(peh_hbm.at[pl.ds(row0, _STAGE_ROWS)],
                        peh_sp.at[pl.ds(row0, _STAGE_ROWS)])

    plsc.subcore_barrier()
    ix.wait()
    xs = []
    for c in range(_N_CHUNKS):
        off = c * _CHUNK
        xs.append(pltpu.async_copy(
            pew_sp.at[idx_x.at[pl.ds(off, _CHUNK)]],
            rows.at[pl.ds(off, _CHUNK)], sx[c]))
    iy.wait()
    ys = []
    for c in range(_N_CHUNKS):
        off = c * _CHUNK
        xs[c].wait()
        ys.append(pltpu.async_copy(
            peh_sp.at[idx_y.at[pl.ds(off, _CHUNK)]],
            rows.at[pl.ds(off, _CHUNK)], sy[c], add=True))
    ws = []
    for c in range(_N_CHUNKS):
        off = c * _CHUNK
        ys[c].wait()
        ws.append(pltpu.async_copy(
            rows.at[pl.ds(off, _CHUNK)],
            out_hbm.at[pl.ds(base + off, _CHUNK)], sw))
    for w in ws:
        w.wait()


@jax.jit
def _pe_kernel(x, y, peh_wide, pew_wide):
    mesh = plsc.VectorSubcoreMesh(core_axis_name="c", subcore_axis_name="s")
    return pl.kernel(
        _pe_body,
        out_type=jax.ShapeDtypeStruct((BATCH, D), jnp.float32),
        mesh=mesh,
        scratch_types=(
            [
                pltpu.VMEM((_B_PER_W,), jnp.int32),
                pltpu.VMEM((_B_PER_W,), jnp.int32),
                pltpu.VMEM((_B_PER_W, D), jnp.float32),
                pltpu.VMEM_SHARED((TABLE_ROWS, D), jnp.float32),
                pltpu.VMEM_SHARED((TABLE_ROWS, D), jnp.float32),
            ]
            + [pltpu.SemaphoreType.DMA] * (2 + 2 * _N_CHUNKS + 1)
        ),
    )(x, y, peh_wide, pew_wide)


def kernel(x, y, pe_h, pe_w):
    x = x.astype(jnp.int32)
    y = y.astype(jnp.int32)
    zeros = jnp.zeros_like(pe_w)
    pew_wide = jnp.concatenate([pe_w, zeros], axis=1)
    peh_wide = jnp.concatenate([zeros, pe_h], axis=1)
    return _pe_kernel(x, y, peh_wide, pew_wide)
